# Initial kernel scaffold; baseline (speedup 1.0000x reference)
import jax
import jax.numpy as jnp
from jax.experimental import pallas as pl


def kernel(lprobs, scores, step):
    raise NotImplementedError("write your pallas kernel here")


# speedup vs baseline: 23.5299x; 23.5299x over previous
import functools

import jax
import jax.numpy as jnp
from jax import lax
from jax.experimental import pallas as pl
from jax.experimental.pallas import tpu as pltpu
from jax.experimental.pallas import tpu_sc as plsc

NC = 2
NS = 16
L = 16
CHUNK = 20000
CAP = 2048
NEG = float("-inf")
IMAX = 2**31 - 1


def _topk16_sc(bsz, beam, vocab):
    row = beam * vocab
    n_chunks = vocab // CHUNK
    mesh = plsc.VectorSubcoreMesh(core_axis_name="c", subcore_axis_name="s")

    @functools.partial(
        pl.kernel,
        mesh=mesh,
        compiler_params=pltpu.CompilerParams(needs_layout_passes=False),
        out_type=(
            jax.ShapeDtypeStruct((bsz, L), jnp.float32),
            jax.ShapeDtypeStruct((bsz, L), jnp.int32),
        ),
        scratch_types=[
            pltpu.VMEM((CHUNK,), jnp.float32),
            pltpu.VMEM((beam * L,), jnp.float32),
            pltpu.VMEM((CAP,), jnp.float32),
            pltpu.VMEM((CAP,), jnp.int32),
            pltpu.VMEM((L,), jnp.float32),
            pltpu.VMEM((L,), jnp.int32),
        ],
    )
    def k(lp_hbm, bias_hbm, out_v_hbm, out_i_hbm,
          chunk, biasv, cand_v, cand_i, obuf_v, obuf_i):
        wid = lax.axis_index("s") * NC + lax.axis_index("c")
        iota = lax.iota(jnp.int32, L)
        neg = jnp.full((L,), NEG, jnp.float32)

        pltpu.sync_copy(bias_hbm.at[wid], biasv)

        def init_body(i, _):
            cand_v[pl.ds(i * L, L)] = neg
            return 0
        lax.fori_loop(0, CAP // L, init_body, 0)

        m = neg
        offv = jnp.zeros((L,), jnp.int32)
        for j in range(beam):
            bvec = biasv[pl.ds(j * L, L)]

            def chunk_body(c, carry, j=j, bvec=bvec):
                m, offv = carry
                start = wid * row + j * vocab + c * CHUNK
                start = pl.multiple_of(start, 8)
                pltpu.sync_copy(lp_hbm.at[pl.ds(start, CHUNK)], chunk)

                def a_body(i, m):
                    v = chunk[pl.ds(i * L, L)] + bvec
                    return jnp.maximum(m, v)
                m = lax.fori_loop(0, CHUNK // L, a_body, m)

                taus = jnp.full((L,), -jnp.max(-m), jnp.float32)
                fbase = j * vocab + c * CHUNK

                def b_body(i, bc):
                    offv, idxv = bc
                    v = chunk[pl.ds(i * L, L)] + bvec
                    msk = v >= taus
                    rank = jnp.cumsum(msk.astype(jnp.int32))
                    pos = jnp.clip(offv + rank - 1, 0, CAP - 1)
                    plsc.store_scatter(cand_v, [pos], v, mask=msk)
                    plsc.store_scatter(cand_i, [pos], idxv, mask=msk)
                    offv = offv + plsc.all_reduce_population_count(msk)
                    return (offv, idxv + L)
                idxv0 = jnp.full((L,), fbase, jnp.int32) + iota
                (offv, _) = lax.fori_loop(0, CHUNK // L, b_body, (offv, idxv0))
                return (m, offv)

            (m, offv) = lax.fori_loop(0, n_chunks, chunk_body, (m, offv))

        nv = jnp.minimum((jnp.max(offv) + L - 1) // L, CAP // L)
        sel_v = neg
        sel_i = jnp.zeros((L,), jnp.int32)
        for t in range(L):
            def p1(i, mm):
                return jnp.maximum(mm, cand_v[pl.ds(i * L, L)])
            ss = jnp.full((L,), jnp.max(lax.fori_loop(0, nv, p1, neg)),
                          jnp.float32)

            def p2(i, ii):
                v = cand_v[pl.ds(i * L, L)]
                ix = cand_i[pl.ds(i * L, L)]
                return jnp.minimum(ii, jnp.where(v == ss, ix, IMAX))
            imaxv = jnp.full((L,), IMAX, jnp.int32)
            isplat = jnp.full((L,), -jnp.max(-lax.fori_loop(0, nv, p2, imaxv)),
                              jnp.int32)

            def p3(i, _):
                v = cand_v[pl.ds(i * L, L)]
                ix = cand_i[pl.ds(i * L, L)]
                cand_v[pl.ds(i * L, L)] = jnp.where(ix == isplat, neg, v)
                return 0
            lax.fori_loop(0, nv, p3, 0)

            sel_v = jnp.where(iota == t, ss, sel_v)
            sel_i = jnp.where(iota == t, isplat, sel_i)

        obuf_v[...] = sel_v
        obuf_i[...] = sel_i
        pltpu.sync_copy(obuf_v, out_v_hbm.at[wid])
        pltpu.sync_copy(obuf_i, out_i_hbm.at[wid])

    return k


def kernel(lprobs, scores, step):
    bsz, beam, vocab = lprobs.shape
    bias = lax.dynamic_index_in_dim(scores, step - 1, axis=2, keepdims=False)
    bias_bcast = jnp.broadcast_to(
        bias[:, :, None], (bsz, beam, L)).reshape(bsz, beam * L)
    lp_flat = lprobs.reshape(bsz * beam * vocab)
    out_v, out_i = _topk16_sc(bsz, beam, vocab)(lp_flat, bias_bcast)
    return out_v, out_i % vocab, out_i // vocab


# speedup vs baseline: 24.6282x; 1.0467x over previous
import functools

import jax
import jax.numpy as jnp
from jax import lax
from jax.experimental import pallas as pl
from jax.experimental.pallas import tpu as pltpu
from jax.experimental.pallas import tpu_sc as plsc

NC = 2
NS = 16
L = 16
CHUNK = 20000
CAP = 2048
NEG = float("-inf")
IMAX = 2**31 - 1


def _topk16_sc(bsz, beam, vocab):
    row = beam * vocab
    n_chunks = vocab // CHUNK
    mesh = plsc.VectorSubcoreMesh(core_axis_name="c", subcore_axis_name="s")

    @functools.partial(
        pl.kernel,
        mesh=mesh,
        compiler_params=pltpu.CompilerParams(needs_layout_passes=False),
        out_type=(
            jax.ShapeDtypeStruct((bsz, L), jnp.float32),
            jax.ShapeDtypeStruct((bsz, L), jnp.int32),
        ),
        scratch_types=[
            pltpu.VMEM((2 * CHUNK,), jnp.float32),
            pltpu.VMEM((beam * L,), jnp.float32),
            pltpu.VMEM((CAP,), jnp.float32),
            pltpu.VMEM((CAP,), jnp.int32),
            pltpu.VMEM((L,), jnp.float32),
            pltpu.VMEM((L,), jnp.int32),
            pltpu.SemaphoreType.DMA,
            pltpu.SemaphoreType.DMA,
        ],
    )
    def k(lp_hbm, bias_hbm, out_v_hbm, out_i_hbm,
          chunk2, biasv, cand_v, cand_i, obuf_v, obuf_i, sem0, sem1):
        wid = lax.axis_index("s") * NC + lax.axis_index("c")
        iota = lax.iota(jnp.int32, L)
        neg = jnp.full((L,), NEG, jnp.float32)

        pltpu.sync_copy(bias_hbm.at[wid], biasv)

        def init_body(i, _):
            cand_v[pl.ds(i * L, L)] = neg
            return 0
        lax.fori_loop(0, CAP // L, init_body, 0)

        m = neg
        offv = jnp.zeros((L,), jnp.int32)
        total = beam * n_chunks
        sems = [sem0, sem1]

        def issue(ci):
            start = pl.multiple_of(wid * row + ci * CHUNK, 8)
            return pltpu.async_copy(
                lp_hbm.at[pl.ds(start, CHUNK)],
                chunk2.at[pl.ds((ci % 2) * CHUNK, CHUNK)], sems[ci % 2])

        handles = [issue(0), None]
        for ci in range(total):
            if ci + 1 < total:
                handles[(ci + 1) % 2] = issue(ci + 1)
            handles[ci % 2].wait()
            buf = chunk2.at[pl.ds((ci % 2) * CHUNK, CHUNK)]
            bvec = biasv[pl.ds((ci // n_chunks) * L, L)]

            def a_body(i, m, buf=buf, bvec=bvec):
                v = buf[pl.ds(i * L, L)] + bvec
                return jnp.maximum(m, v)
            m = lax.fori_loop(0, CHUNK // L, a_body, m)

            taus = jnp.full((L,), -jnp.max(-m), jnp.float32)

            def b_body(i, bc, buf=buf, bvec=bvec, taus=taus):
                offv, idxv = bc
                v = buf[pl.ds(i * L, L)] + bvec
                msk = v >= taus
                rank = jnp.cumsum(msk.astype(jnp.int32))
                pos = jnp.clip(offv + rank - 1, 0, CAP - 1)
                plsc.store_scatter(cand_v, [pos], v, mask=msk)
                plsc.store_scatter(cand_i, [pos], idxv, mask=msk)
                offv = offv + plsc.all_reduce_population_count(msk)
                return (offv, idxv + L)
            idxv0 = jnp.full((L,), ci * CHUNK, jnp.int32) + iota
            (offv, _) = lax.fori_loop(0, CHUNK // L, b_body, (offv, idxv0))

        nv = jnp.minimum((jnp.max(offv) + L - 1) // L, CAP // L)
        sel_v = neg
        sel_i = jnp.zeros((L,), jnp.int32)
        for t in range(L):
            def p1(i, mm):
                return jnp.maximum(mm, cand_v[pl.ds(i * L, L)])
            ss = jnp.full((L,), jnp.max(lax.fori_loop(0, nv, p1, neg)),
                          jnp.float32)

            def p2(i, ii):
                v = cand_v[pl.ds(i * L, L)]
                ix = cand_i[pl.ds(i * L, L)]
                return jnp.minimum(ii, jnp.where(v == ss, ix, IMAX))
            imaxv = jnp.full((L,), IMAX, jnp.int32)
            isplat = jnp.full((L,), -jnp.max(-lax.fori_loop(0, nv, p2, imaxv)),
                              jnp.int32)

            def p3(i, _):
                v = cand_v[pl.ds(i * L, L)]
                ix = cand_i[pl.ds(i * L, L)]
                cand_v[pl.ds(i * L, L)] = jnp.where(ix == isplat, neg, v)
                return 0
            lax.fori_loop(0, nv, p3, 0)

            sel_v = jnp.where(iota == t, ss, sel_v)
            sel_i = jnp.where(iota == t, isplat, sel_i)

        obuf_v[...] = sel_v
        obuf_i[...] = sel_i
        pltpu.sync_copy(obuf_v, out_v_hbm.at[wid])
        pltpu.sync_copy(obuf_i, out_i_hbm.at[wid])

    return k


def kernel(lprobs, scores, step):
    bsz, beam, vocab = lprobs.shape
    bias = lax.dynamic_index_in_dim(scores, step - 1, axis=2, keepdims=False)
    bias_bcast = jnp.broadcast_to(
        bias[:, :, None], (bsz, beam, L)).reshape(bsz, beam * L)
    lp_flat = lprobs.reshape(bsz * beam * vocab)
    out_v, out_i = _topk16_sc(bsz, beam, vocab)(lp_flat, bias_bcast)
    return out_v, out_i % vocab, out_i // vocab


# speedup vs baseline: 60.3796x; 2.4516x over previous
import functools

import jax
import jax.numpy as jnp
from jax import lax
from jax.experimental import pallas as pl
from jax.experimental.pallas import tpu as pltpu
from jax.experimental.pallas import tpu_sc as plsc

NC = 2
NS = 16
L = 16
CHUNK = 20000
GV = 50
UN = 10
CAP = 16384
NEG = float("-inf")
IMAX = 2**31 - 1


def _topk16_sc(bsz, beam, vocab):
    row = beam * vocab
    total_chunks = row // CHUNK
    totaln = bsz * row
    mesh = plsc.VectorSubcoreMesh(core_axis_name="c", subcore_axis_name="s")

    @functools.partial(
        pl.kernel,
        mesh=mesh,
        compiler_params=pltpu.CompilerParams(needs_layout_passes=False),
        out_type=(
            jax.ShapeDtypeStruct((bsz, L), jnp.float32),
            jax.ShapeDtypeStruct((bsz, L), jnp.int32),
        ),
        scratch_types=[
            pltpu.VMEM((2 * CHUNK,), jnp.float32),
            pltpu.VMEM((beam * L,), jnp.float32),
            pltpu.VMEM((CAP,), jnp.float32),
            pltpu.VMEM((CAP,), jnp.int32),
            pltpu.VMEM((L,), jnp.float32),
            pltpu.VMEM((L,), jnp.int32),
            pltpu.SemaphoreType.DMA,
            pltpu.SemaphoreType.DMA,
        ],
    )
    def k(lp_hbm, bias_hbm, out_v_hbm, out_i_hbm,
          chunk2, biasv, cand_v, cand_i, obuf_v, obuf_i, sem0, sem1):
        wid = lax.axis_index("s") * NC + lax.axis_index("c")
        iota = lax.iota(jnp.int32, L)
        neg = jnp.full((L,), NEG, jnp.float32)
        zero_i = jnp.zeros((L,), jnp.int32)
        sems = [sem0, sem1]

        pltpu.sync_copy(bias_hbm.at[wid], biasv)

        def init_body(i, _):
            cand_v[pl.ds(i * L, L)] = neg
            return 0
        lax.fori_loop(0, CAP // L, init_body, 0)

        def cp(ci, b):
            start = jnp.minimum(wid * row + ci * CHUNK, totaln - CHUNK)
            return pltpu.async_copy(
                lp_hbm.at[pl.ds(pl.multiple_of(start, 8), CHUNK)],
                chunk2.at[pl.ds(b * CHUNK, CHUNK)], sems[b])

        def wait(b):
            pltpu.make_async_copy(
                lp_hbm.at[pl.ds(0, CHUNK)],
                chunk2.at[pl.ds(b * CHUNK, CHUNK)], sems[b]).wait()

        def chunk_compute(ci, b, m, offv):
            wait(b)
            boff = pl.multiple_of((ci // (vocab // CHUNK)) * L, 8)
            bvec = biasv[pl.ds(boff, L)]

            def group_body(g, carry, b=b, bvec=bvec):
                m, offv = carry
                gb = pl.multiple_of(b * CHUNK + g * (GV * L), 8)

                def sb(t, acc, gb=gb, bvec=bvec):
                    base = pl.multiple_of(gb + t * (UN * L), 8)
                    for u in range(UN):
                        acc = jnp.maximum(
                            acc, chunk2[pl.ds(base + u * L, L)] + bvec)
                    return acc
                acc = lax.fori_loop(0, GV // UN, sb, neg)
                m = jnp.maximum(m, acc)
                taus = jnp.full((L,), -jnp.max(-m), jnp.float32)
                cnt = jnp.max(
                    plsc.all_reduce_population_count(acc >= taus))

                def do_filter(off, gb=gb, bvec=bvec, taus=taus):
                    fb = ci * CHUNK + g * (GV * L)

                    def fbdy(t, off, gb=gb, fb=fb, bvec=bvec, taus=taus):
                        base = pl.multiple_of(gb + t * (UN * L), 8)
                        ib = fb + t * (UN * L)
                        for u in range(UN):
                            v = chunk2[pl.ds(base + u * L, L)] + bvec
                            anyb = plsc.all_reduce_population_count(
                                v >= taus) > 0
                            pos = jnp.minimum(off + iota, CAP - 1)
                            idxv = jnp.full((L,), ib + u * L, jnp.int32) + iota
                            plsc.store_scatter(cand_v, [pos], v, mask=anyb)
                            plsc.store_scatter(cand_i, [pos], idxv, mask=anyb)
                            off = off + jnp.where(anyb, L, 0)
                        return off
                    return lax.fori_loop(0, GV // UN, fbdy, off)

                offv = lax.cond(cnt > 0, do_filter, lambda o: o, offv)
                return (m, offv)

            return lax.fori_loop(0, CHUNK // (GV * L), group_body, (m, offv))

        cp(0, 0)

        def pair_body(p, carry):
            m, offv = carry
            for b in range(2):
                ci = 2 * p + b
                cp(ci + 1, (b + 1) % 2)
                m, offv = chunk_compute(ci, b, m, offv)
            return (m, offv)
        m, offv = lax.fori_loop(
            0, total_chunks // 2, pair_body,
            (neg, zero_i))
        wait(0)

        taus = jnp.full((L,), -jnp.max(-m), jnp.float32)
        nv = jnp.minimum((jnp.max(offv) + L - 1) // L, CAP // L)

        def comp_body(i, o2):
            v = cand_v[pl.ds(i * L, L)]
            ix = cand_i[pl.ds(i * L, L)]
            msk = v >= taus
            rank = jnp.cumsum(msk.astype(jnp.int32))
            pos = jnp.clip(o2 + rank - 1, 0, CAP - 1)
            plsc.store_scatter(cand_v, [pos], v, mask=msk)
            plsc.store_scatter(cand_i, [pos], ix, mask=msk)
            return o2 + plsc.all_reduce_population_count(msk)
        off2 = lax.fori_loop(0, nv, comp_body, zero_i)
        n2 = jnp.max(off2)
        nv2 = jnp.minimum((n2 + L - 1) // L, CAP // L)
        pm = (off2 + iota) < jnp.full((L,), nv2 * L, jnp.int32)
        ppos = jnp.minimum(off2 + iota, CAP - 1)
        plsc.store_scatter(cand_v, [ppos], neg, mask=pm)

        sel_v = neg
        sel_i = zero_i
        for t in range(L):
            def p1(i, mm):
                return jnp.maximum(mm, cand_v[pl.ds(i * L, L)])
            ss = jnp.full((L,), jnp.max(lax.fori_loop(0, nv2, p1, neg)),
                          jnp.float32)

            def p2(i, ii):
                v = cand_v[pl.ds(i * L, L)]
                ix = cand_i[pl.ds(i * L, L)]
                return jnp.minimum(ii, jnp.where(v == ss, ix, IMAX))
            imaxv = jnp.full((L,), IMAX, jnp.int32)
            isplat = jnp.full((L,), -jnp.max(-lax.fori_loop(0, nv2, p2, imaxv)),
                              jnp.int32)

            def p3(i, _):
                v = cand_v[pl.ds(i * L, L)]
                ix = cand_i[pl.ds(i * L, L)]
                cand_v[pl.ds(i * L, L)] = jnp.where(ix == isplat, neg, v)
                return 0
            lax.fori_loop(0, nv2, p3, 0)

            sel_v = jnp.where(iota == t, ss, sel_v)
            sel_i = jnp.where(iota == t, isplat, sel_i)

        obuf_v[...] = sel_v
        obuf_i[...] = sel_i
        pltpu.sync_copy(obuf_v, out_v_hbm.at[wid])
        pltpu.sync_copy(obuf_i, out_i_hbm.at[wid])

    return k


def kernel(lprobs, scores, step):
    bsz, beam, vocab = lprobs.shape
    bias = lax.dynamic_index_in_dim(scores, step - 1, axis=2, keepdims=False)
    bias_bcast = jnp.broadcast_to(
        bias[:, :, None], (bsz, beam, L)).reshape(bsz, beam * L)
    lp_flat = lprobs.reshape(bsz * beam * vocab)
    out_v, out_i = _topk16_sc(bsz, beam, vocab)(lp_flat, bias_bcast)
    return out_v, out_i % vocab, out_i // vocab


# speedup vs baseline: 62.5938x; 1.0367x over previous
import functools

import jax
import jax.numpy as jnp
from jax import lax
from jax.experimental import pallas as pl
from jax.experimental.pallas import tpu as pltpu
from jax.experimental.pallas import tpu_sc as plsc

NC = 2
NS = 16
L = 16
CHUNK = 20000
GV = 50
UN = 10
CAP = 16384
NEG = float("-inf")
IMAX = 2**31 - 1


def _topk16_sc(bsz, beam, vocab):
    row = beam * vocab
    total_chunks = row // CHUNK
    totaln = bsz * row
    mesh = plsc.VectorSubcoreMesh(core_axis_name="c", subcore_axis_name="s")

    @functools.partial(
        pl.kernel,
        mesh=mesh,
        compiler_params=pltpu.CompilerParams(needs_layout_passes=False),
        out_type=(
            jax.ShapeDtypeStruct((bsz, L), jnp.float32),
            jax.ShapeDtypeStruct((bsz, L), jnp.int32),
        ),
        scratch_types=[
            pltpu.VMEM((2 * CHUNK,), jnp.float32),
            pltpu.VMEM((beam * L,), jnp.float32),
            pltpu.VMEM((CAP,), jnp.float32),
            pltpu.VMEM((CAP,), jnp.int32),
            pltpu.VMEM((L,), jnp.float32),
            pltpu.VMEM((L,), jnp.int32),
            pltpu.SemaphoreType.DMA,
            pltpu.SemaphoreType.DMA,
        ],
    )
    def k(lp_hbm, bias_hbm, out_v_hbm, out_i_hbm,
          chunk2, biasv, cand_v, cand_i, obuf_v, obuf_i, sem0, sem1):
        wid = lax.axis_index("s") * NC + lax.axis_index("c")
        iota = lax.iota(jnp.int32, L)
        neg = jnp.full((L,), NEG, jnp.float32)
        zero_i = jnp.zeros((L,), jnp.int32)
        sems = [sem0, sem1]

        pltpu.sync_copy(bias_hbm.at[wid], biasv)

        def init_body(i, _):
            cand_v[pl.ds(i * L, L)] = neg
            return 0
        lax.fori_loop(0, CAP // L, init_body, 0)

        def cp(ci, b):
            start = jnp.minimum(wid * row + ci * CHUNK, totaln - CHUNK)
            return pltpu.async_copy(
                lp_hbm.at[pl.ds(pl.multiple_of(start, 8), CHUNK)],
                chunk2.at[pl.ds(b * CHUNK, CHUNK)], sems[b])

        def wait(b):
            pltpu.make_async_copy(
                lp_hbm.at[pl.ds(0, CHUNK)],
                chunk2.at[pl.ds(b * CHUNK, CHUNK)], sems[b]).wait()

        def chunk_compute(ci, b, m, offv, taus):
            wait(b)
            boff = pl.multiple_of((ci // (vocab // CHUNK)) * L, 8)
            bvec = biasv[pl.ds(boff, L)]

            def group_body(g, carry, b=b, bvec=bvec):
                m, offv, taus = carry
                gb = pl.multiple_of(b * CHUNK + g * (GV * L), 8)

                def sb(t, accs, gb=gb):
                    base = pl.multiple_of(gb + t * (UN * L), 8)
                    accs = list(accs)
                    for u in range(UN):
                        accs[u % 4] = jnp.maximum(
                            accs[u % 4], chunk2[pl.ds(base + u * L, L)])
                    return tuple(accs)
                a0, a1, a2, a3 = lax.fori_loop(
                    0, GV // UN, sb, (neg, neg, neg, neg))
                acc = jnp.maximum(jnp.maximum(a0, a1), jnp.maximum(a2, a3))
                m = jnp.maximum(m, acc + bvec)
                cnt = plsc.all_reduce_population_count(
                    acc >= (taus - bvec))[0]

                def do_filter(args, gb=gb, bvec=bvec, m=m):
                    off, _ = args
                    taus = jnp.full((L,), -jnp.max(-m), jnp.float32)
                    fb = ci * CHUNK + g * (GV * L)

                    def fbdy(t, off, gb=gb, fb=fb, bvec=bvec, taus=taus):
                        base = pl.multiple_of(gb + t * (UN * L), 8)
                        ib = fb + t * (UN * L)
                        for u in range(UN):
                            v = chunk2[pl.ds(base + u * L, L)] + bvec
                            anyb = plsc.all_reduce_population_count(
                                v >= taus) > 0
                            pos = jnp.minimum(off + iota, CAP - 1)
                            idxv = jnp.full((L,), ib + u * L, jnp.int32) + iota
                            plsc.store_scatter(cand_v, [pos], v, mask=anyb)
                            plsc.store_scatter(cand_i, [pos], idxv, mask=anyb)
                            off = off + jnp.where(anyb, L, 0)
                        return off
                    return (lax.fori_loop(0, GV // UN, fbdy, off), taus)

                offv, taus = lax.cond(
                    cnt > 0, do_filter, lambda a: a, (offv, taus))
                return (m, offv, taus)

            return lax.fori_loop(
                0, CHUNK // (GV * L), group_body, (m, offv, taus))

        cp(0, 0)

        def pair_body(p, carry):
            m, offv, taus = carry
            for b in range(2):
                ci = 2 * p + b
                cp(ci + 1, (b + 1) % 2)
                m, offv, taus = chunk_compute(ci, b, m, offv, taus)
            return (m, offv, taus)
        m, offv, _ = lax.fori_loop(
            0, total_chunks // 2, pair_body,
            (neg, zero_i, neg))
        wait(0)

        taus = jnp.full((L,), -jnp.max(-m), jnp.float32)
        nv = jnp.minimum((jnp.max(offv) + L - 1) // L, CAP // L)

        def comp_body(i, o2):
            v = cand_v[pl.ds(i * L, L)]
            ix = cand_i[pl.ds(i * L, L)]
            msk = v >= taus
            rank = jnp.cumsum(msk.astype(jnp.int32))
            pos = jnp.clip(o2 + rank - 1, 0, CAP - 1)
            plsc.store_scatter(cand_v, [pos], v, mask=msk)
            plsc.store_scatter(cand_i, [pos], ix, mask=msk)
            return o2 + plsc.all_reduce_population_count(msk)
        off2 = lax.fori_loop(0, nv, comp_body, zero_i)
        n2 = jnp.max(off2)
        nv2 = jnp.minimum((n2 + L - 1) // L, CAP // L)
        pm = (off2 + iota) < jnp.full((L,), nv2 * L, jnp.int32)
        ppos = jnp.minimum(off2 + iota, CAP - 1)
        plsc.store_scatter(cand_v, [ppos], neg, mask=pm)

        sel_v = neg
        sel_i = zero_i
        for t in range(L):
            def p1(i, mm):
                return jnp.maximum(mm, cand_v[pl.ds(i * L, L)])
            ss = jnp.full((L,), jnp.max(lax.fori_loop(0, nv2, p1, neg)),
                          jnp.float32)

            def p2(i, ii):
                v = cand_v[pl.ds(i * L, L)]
                ix = cand_i[pl.ds(i * L, L)]
                return jnp.minimum(ii, jnp.where(v == ss, ix, IMAX))
            imaxv = jnp.full((L,), IMAX, jnp.int32)
            isplat = jnp.full((L,), -jnp.max(-lax.fori_loop(0, nv2, p2, imaxv)),
                              jnp.int32)

            def p3(i, _):
                v = cand_v[pl.ds(i * L, L)]
                ix = cand_i[pl.ds(i * L, L)]
                cand_v[pl.ds(i * L, L)] = jnp.where(ix == isplat, neg, v)
                return 0
            lax.fori_loop(0, nv2, p3, 0)

            sel_v = jnp.where(iota == t, ss, sel_v)
            sel_i = jnp.where(iota == t, isplat, sel_i)

        obuf_v[...] = sel_v
        obuf_i[...] = sel_i
        pltpu.sync_copy(obuf_v, out_v_hbm.at[wid])
        pltpu.sync_copy(obuf_i, out_i_hbm.at[wid])

    return k


def kernel(lprobs, scores, step):
    bsz, beam, vocab = lprobs.shape
    bias = lax.dynamic_index_in_dim(scores, step - 1, axis=2, keepdims=False)
    bias_bcast = jnp.broadcast_to(
        bias[:, :, None], (bsz, beam, L)).reshape(bsz, beam * L)
    lp_flat = lprobs.reshape(bsz * beam * vocab)
    out_v, out_i = _topk16_sc(bsz, beam, vocab)(lp_flat, bias_bcast)
    return out_v, out_i % vocab, out_i // vocab
